# baseline (speedup 1.0000x reference)
import functools

import jax
import jax.numpy as jnp
import numpy as np
from jax import lax
from jax.experimental import pallas as pl
from jax.experimental.pallas import tpu as pltpu
from jax.experimental.pallas import tpu_sc as plsc

P_MIN = 0.8
THRESHOLD = 1e-10
L = 16
NC = 2
NW = 32
RSQRT_SEED = 0x5F3759DF
TN = 256


def _sqrt16(s):
    i = lax.bitcast_convert_type(s, jnp.int32)
    g = lax.bitcast_convert_type(
        jnp.int32(RSQRT_SEED) - lax.shift_right_arithmetic(i, 1), jnp.float32)
    g = g * (1.5 - 0.5 * s * g * g)
    g = g * (1.5 - 0.5 * s * g * g)
    g = g * (1.5 - 0.5 * s * g * g)
    return s * g


def _sc_body(s_hbm, out_hbm, buf, d_scr, out_v, *,
             n, m, b, cpw, inv_log_ratio):
    wid = lax.axis_index("s") * NC + lax.axis_index("c")
    pltpu.sync_copy(s_hbm.at[0], buf.at[pl.ds(0, 3 * n)])
    pltpu.sync_copy(s_hbm.at[b], buf.at[pl.ds(3 * n, 3 * n)])
    nchunk_blk = n // L

    def chunk_body(i, loss_v):
        c = wid * cpw + i
        rs = jnp.where(c >= nchunk_blk, 1, 0)
        rhalf = rs * (3 * n)
        chalf = (1 - rs) * (3 * n)
        base = rhalf + (c - rs * nchunk_blk) * L
        ax = buf[pl.ds(base, L)]
        ay = buf[pl.ds(n + base, L)]
        az = buf[pl.ds(2 * n + base, L)]

        def pass1(j, carry):
            m1, m2 = carry
            cvx = buf[pl.ds(chalf + j * L, L)]
            cvy = buf[pl.ds(chalf + m + j * L, L)]
            cvz = buf[pl.ds(chalf + 2 * m + j * L, L)]
            for k in range(L):
                dx = ax - cvx[k]
                dy = ay - cvy[k]
                dz = az - cvz[k]
                s = dx * dx + dy * dy + dz * dz
                d_scr[pl.ds((j * L + k) * L, L)] = s
                m2 = jnp.minimum(m2, jnp.maximum(m1, s))
                m1 = jnp.minimum(m1, s)
            return m1, m2

        big = jnp.full((L,), 3.0e38, jnp.float32)
        m1, m2 = lax.fori_loop(0, m // L, pass1, (big, big))
        d1 = jnp.maximum(_sqrt16(m1), 1e-6)
        d2 = jnp.maximum(_sqrt16(m2), 1e-6)
        it = 1.0 / jnp.maximum((d2 - d1) * inv_log_ratio, 1e-6)

        def pass2(j, carry):
            e_acc, ed_acc = carry
            for k in range(L):
                d = jnp.maximum(_sqrt16(d_scr[pl.ds((j * L + k) * L, L)]),
                                1e-6)
                e = jnp.exp((d1 - d) * it)
                e_acc = e_acc + e
                ed_acc = ed_acc + e * d
            return e_acc, ed_acc

        zero = jnp.zeros((L,), jnp.float32)
        e_acc, ed_acc = lax.fori_loop(0, m // L, pass2, (zero, zero))
        return loss_v + ed_acc / e_acc

    loss_v = lax.fori_loop(0, cpw, chunk_body, jnp.zeros((L,), jnp.float32))
    out_v[...] = loss_v
    pltpu.sync_copy(out_v, out_hbm.at[wid])


def _tc_body(a_ref, c_ref, o_ref, *, m, log_ratio):
    a = a_ref[0]
    c = c_ref[0]
    a0, a1, a2 = a[:, 0:1], a[:, 1:2], a[:, 2:3]
    tn = a.shape[0]
    mc = 128
    big = 3.0e38

    m1v = jnp.full((tn, mc), big, jnp.float32)
    m2v = m1v
    chunks = []
    for k in range(m // mc):
        ck = c[:, k * mc:(k + 1) * mc]
        sc_ = jnp.maximum(
            (a0 - ck[0:1, :]) ** 2
            + (a1 - ck[1:2, :]) ** 2
            + (a2 - ck[2:3, :]) ** 2,
            1e-12)
        m2v = jnp.minimum(m2v, jnp.maximum(m1v, sc_))
        m1v = jnp.minimum(m1v, sc_)
        chunks.append(sc_)
    s = jnp.concatenate(chunks, axis=1)

    m1 = jnp.min(m1v, axis=1, keepdims=True)
    eq = m1v == m1
    cnt = jnp.sum(eq.astype(jnp.float32), axis=1, keepdims=True)
    m2d = jnp.min(jnp.where(eq, big, m1v), axis=1, keepdims=True)
    m2c = jnp.min(m2v, axis=1, keepdims=True)
    m2 = jnp.where(cnt > 1.0, m1, jnp.minimum(m2d, m2c))

    d1 = jnp.maximum(jnp.sqrt(m1), 1e-6)
    d2 = jnp.maximum(jnp.sqrt(m2), 1e-6)
    it = 1.0 / jnp.maximum((d2 - d1) * (1.0 / log_ratio), 1e-6)

    eacc = jnp.zeros((tn, mc), jnp.float32)
    edacc = eacc
    for k in range(m // mc):
        sk = s[:, k * mc:(k + 1) * mc]
        dk = jnp.maximum(sk * jax.lax.rsqrt(sk), 1e-6)
        ek = jnp.exp((d1 - dk) * it)
        eacc = eacc + ek
        edacc = edacc + ek * dk
    esum = jnp.sum(eacc, axis=1, keepdims=True)
    srow = jnp.sum(edacc, axis=1, keepdims=True)
    step = jnp.sum(srow / esum, keepdims=True).reshape(1, 1)

    @pl.when(pl.program_id(0) == 0)
    def _init():
        o_ref[0] = jnp.zeros((1, 1), jnp.float32)

    o_ref[0] += step


def kernel(x, y):
    b, n, _ = x.shape
    m = y.shape[1]
    nblk = 2 * b
    log_ratio = float(np.log(P_MIN * (m - 1) / (1.0 - P_MIN)))

    a_all = jnp.concatenate([x, y], axis=0)
    c_all = jnp.concatenate([y, x], axis=0).transpose(0, 2, 1)
    s_all = a_all.transpose(0, 2, 1).reshape(nblk, 3 * n)

    cpw = 7
    sc_run = pl.kernel(
        functools.partial(_sc_body, n=n, m=m, b=b, cpw=cpw,
                          inv_log_ratio=1.0 / log_ratio),
        out_type=jax.ShapeDtypeStruct((NW, L), jnp.float32),
        mesh=plsc.VectorSubcoreMesh(core_axis_name="c", subcore_axis_name="s"),
        scratch_types=[
            pltpu.VMEM((2 * 3 * n,), jnp.float32),
            pltpu.VMEM((m * L,), jnp.float32),
            pltpu.VMEM((L,), jnp.float32),
        ],
    )
    sc_partial = sc_run(s_all)

    nt = n // TN
    chunks_per_tile = TN // L
    b_tail_from = (NW * cpw - n // L) // chunks_per_tile
    full_steps = (nblk - 2) * nt
    steps = full_steps + nt - b_tail_from

    def blk_of(i):
        q = i // nt
        full_blk = jnp.where(q >= b - 1, q + 2, q + 1)
        return jnp.where(i < full_steps, full_blk, b)

    def tile_of(i):
        return jnp.where(i < full_steps, i % nt, i - full_steps + b_tail_from)

    tc_partial = pl.pallas_call(
        functools.partial(_tc_body, m=m, log_ratio=log_ratio),
        grid=(steps,),
        in_specs=[
            pl.BlockSpec((1, TN, 3), lambda i: (blk_of(i), tile_of(i), 0)),
            pl.BlockSpec((1, 3, m), lambda i: (blk_of(i), 0, 0)),
        ],
        out_specs=pl.BlockSpec((1, 1, 1), lambda i: (0, 0, 0)),
        out_shape=jax.ShapeDtypeStruct((1, 1, 1), jnp.float32),
    )(a_all, c_all)

    return jnp.sum(sc_partial) + tc_partial[0, 0, 0]


# speedup vs baseline: 1.0921x; 1.0921x over previous
import functools

import jax
import jax.numpy as jnp
import numpy as np
from jax import lax
from jax.experimental import pallas as pl
from jax.experimental.pallas import tpu as pltpu
from jax.experimental.pallas import tpu_sc as plsc

P_MIN = 0.8
THRESHOLD = 1e-10
L = 16
NC = 2
NW = 32
RSQRT_SEED = 0x5F3759DF
TN = 256


def _sqrt16(s):
    i = lax.bitcast_convert_type(s, jnp.int32)
    g = lax.bitcast_convert_type(
        jnp.int32(RSQRT_SEED) - lax.shift_right_arithmetic(i, 1), jnp.float32)
    g = g * (1.5 - 0.5 * s * g * g)
    g = g * (1.5 - 0.5 * s * g * g)
    g = g * (1.5 - 0.5 * s * g * g)
    return s * g


def _sc_body(s_hbm, out_hbm, buf, d_scr, out_v, *,
             n, m, b, cpw, inv_log_ratio):
    wid = lax.axis_index("s") * NC + lax.axis_index("c")
    pltpu.sync_copy(s_hbm.at[0], buf.at[pl.ds(0, 3 * n)])
    pltpu.sync_copy(s_hbm.at[b], buf.at[pl.ds(3 * n, 3 * n)])
    nchunk_blk = n // L

    def chunk_body(i, loss_v):
        c = wid * cpw + i
        rs = jnp.where(c >= nchunk_blk, 1, 0)
        rhalf = rs * (3 * n)
        chalf = (1 - rs) * (3 * n)
        base = rhalf + (c - rs * nchunk_blk) * L
        ax = buf[pl.ds(base, L)]
        ay = buf[pl.ds(n + base, L)]
        az = buf[pl.ds(2 * n + base, L)]

        def pass1(j, carry):
            m1, m2 = carry
            cvx = buf[pl.ds(chalf + j * L, L)]
            cvy = buf[pl.ds(chalf + m + j * L, L)]
            cvz = buf[pl.ds(chalf + 2 * m + j * L, L)]
            for k in range(L):
                dx = ax - cvx[k]
                dy = ay - cvy[k]
                dz = az - cvz[k]
                s = dx * dx + dy * dy + dz * dz
                d_scr[pl.ds((j * L + k) * L, L)] = s
                m2 = jnp.minimum(m2, jnp.maximum(m1, s))
                m1 = jnp.minimum(m1, s)
            return m1, m2

        big = jnp.full((L,), 3.0e38, jnp.float32)
        m1, m2 = lax.fori_loop(0, m // L, pass1, (big, big))
        d1 = jnp.maximum(_sqrt16(m1), 1e-6)
        d2 = jnp.maximum(_sqrt16(m2), 1e-6)
        it = 1.0 / jnp.maximum((d2 - d1) * inv_log_ratio, 1e-6)

        def pass2(j, carry):
            e_acc, ed_acc = carry
            for k in range(L):
                d = jnp.maximum(_sqrt16(d_scr[pl.ds((j * L + k) * L, L)]),
                                1e-6)
                e = jnp.exp((d1 - d) * it)
                e_acc = e_acc + e
                ed_acc = ed_acc + e * d
            return e_acc, ed_acc

        zero = jnp.zeros((L,), jnp.float32)
        e_acc, ed_acc = lax.fori_loop(0, m // L, pass2, (zero, zero))
        return loss_v + ed_acc / e_acc

    loss_v = lax.fori_loop(0, cpw, chunk_body, jnp.zeros((L,), jnp.float32))
    out_v[...] = loss_v
    pltpu.sync_copy(out_v, out_hbm.at[wid])


def _tc_body(a_ref, c_ref, o_ref, *, m, log_ratio):
    a = a_ref[0]
    c = c_ref[0]
    a0 = jnp.transpose(a[0:1, :])
    a1 = jnp.transpose(a[1:2, :])
    a2 = jnp.transpose(a[2:3, :])
    tn = a.shape[1]
    mc = 128
    big = 3.0e38

    m1v = jnp.full((tn, mc), big, jnp.float32)
    m2v = m1v
    chunks = []
    for k in range(m // mc):
        ck = c[:, k * mc:(k + 1) * mc]
        sc_ = jnp.maximum(
            (a0 - ck[0:1, :]) ** 2
            + (a1 - ck[1:2, :]) ** 2
            + (a2 - ck[2:3, :]) ** 2,
            1e-12)
        m2v = jnp.minimum(m2v, jnp.maximum(m1v, sc_))
        m1v = jnp.minimum(m1v, sc_)
        chunks.append(sc_)
    s = jnp.concatenate(chunks, axis=1)

    m1 = jnp.min(m1v, axis=1, keepdims=True)
    eq = m1v == m1
    cnt = jnp.sum(eq.astype(jnp.float32), axis=1, keepdims=True)
    m2d = jnp.min(jnp.where(eq, big, m1v), axis=1, keepdims=True)
    m2c = jnp.min(m2v, axis=1, keepdims=True)
    m2 = jnp.where(cnt > 1.0, m1, jnp.minimum(m2d, m2c))

    d1 = jnp.maximum(jnp.sqrt(m1), 1e-6)
    d2 = jnp.maximum(jnp.sqrt(m2), 1e-6)
    it = 1.0 / jnp.maximum((d2 - d1) * (1.0 / log_ratio), 1e-6)

    eacc = jnp.zeros((tn, mc), jnp.float32)
    edacc = eacc
    for k in range(m // mc):
        sk = s[:, k * mc:(k + 1) * mc]
        dk = jnp.maximum(sk * jax.lax.rsqrt(sk), 1e-6)
        ek = jnp.exp((d1 - dk) * it)
        eacc = eacc + ek
        edacc = edacc + ek * dk
    esum = jnp.sum(eacc, axis=1, keepdims=True)
    srow = jnp.sum(edacc, axis=1, keepdims=True)
    step = jnp.sum(srow / esum, keepdims=True).reshape(1, 1)

    @pl.when(pl.program_id(0) == 0)
    def _init():
        o_ref[0] = jnp.zeros((1, 1), jnp.float32)

    o_ref[0] += step


def kernel(x, y):
    b, n, _ = x.shape
    m = y.shape[1]
    nblk = 2 * b
    log_ratio = float(np.log(P_MIN * (m - 1) / (1.0 - P_MIN)))

    s_all3 = jnp.concatenate([x, y], axis=0).transpose(0, 2, 1)
    s_all = s_all3.reshape(nblk, 3 * n)

    cpw = 6
    sc_run = pl.kernel(
        functools.partial(_sc_body, n=n, m=m, b=b, cpw=cpw,
                          inv_log_ratio=1.0 / log_ratio),
        out_type=jax.ShapeDtypeStruct((NW, L), jnp.float32),
        mesh=plsc.VectorSubcoreMesh(core_axis_name="c", subcore_axis_name="s"),
        scratch_types=[
            pltpu.VMEM((2 * 3 * n,), jnp.float32),
            pltpu.VMEM((m * L,), jnp.float32),
            pltpu.VMEM((L,), jnp.float32),
        ],
    )
    sc_partial = sc_run(s_all)

    nt = n // TN
    chunks_per_tile = TN // L
    b_tail_from = (NW * cpw - n // L) // chunks_per_tile
    full_steps = (nblk - 2) * nt
    steps = full_steps + nt - b_tail_from

    def blk_of(i):
        q = i // nt
        full_blk = jnp.where(q >= b - 1, q + 2, q + 1)
        return jnp.where(i < full_steps, full_blk, b)

    def tile_of(i):
        return jnp.where(i < full_steps, i % nt, i - full_steps + b_tail_from)

    tc_partial = pl.pallas_call(
        functools.partial(_tc_body, m=m, log_ratio=log_ratio),
        grid=(steps,),
        in_specs=[
            pl.BlockSpec((1, 3, TN), lambda i: (blk_of(i), 0, tile_of(i))),
            pl.BlockSpec((1, 3, m), lambda i: ((blk_of(i) + b) % nblk, 0, 0)),
        ],
        out_specs=pl.BlockSpec((1, 1, 1), lambda i: (0, 0, 0)),
        out_shape=jax.ShapeDtypeStruct((1, 1, 1), jnp.float32),
    )(s_all3, s_all3)

    return jnp.sum(sc_partial) + tc_partial[0, 0, 0]
